# baseline (speedup 1.0000x reference)
import jax
import jax.numpy as jnp
from jax.experimental import pallas as pl
import jax.experimental.pallas.tpu as pltpu

_BK = 128


def _gcn2_kernel(x_ref, a_ref, w1_ref, b1_ref, w2_ref, b2_ref, out_ref,
                 a_s, deg_s, h_s):
    i = pl.program_id(0)
    n_blocks = pl.num_programs(0)
    a_blk = a_ref[...]

    ones = jnp.ones((a_blk.shape[0], 1), dtype=a_blk.dtype)
    deg_part = jax.lax.dot_general(
        a_blk, ones, (((0,), (0,)), ((), ())),
        preferred_element_type=jnp.float32,
    )

    @pl.when(i == 0)
    def _init():
        deg_s[...] = deg_part + 1.0
        h_s[...] = jnp.dot(x_ref[...], w1_ref[...],
                           preferred_element_type=jnp.float32)

    @pl.when(i > 0)
    def _acc():
        deg_s[...] += deg_part

    a_s[pl.ds(i * _BK, _BK), :] = a_blk

    @pl.when(i == n_blocks - 1)
    def _tail():
        deg = deg_s[...]
        s = jnp.where(deg > 0, jax.lax.rsqrt(deg), 0.0)
        a = a_s[...]

        def layer(h, b_ref):
            hs = s * h
            m = jax.lax.dot_general(
                a, hs, (((0,), (0,)), ((), ())),
                preferred_element_type=jnp.float32,
            ) + hs
            return jax.nn.relu(s * m + b_ref[...])

        h1 = layer(h_s[...], b1_ref)
        h2 = jnp.dot(h1, w2_ref[...], preferred_element_type=jnp.float32)
        out_ref[...] = layer(h2, b2_ref)


def kernel(x, adjacency_matrix, W1, b1, W2, b2):
    n, d_out = x.shape[0], W2.shape[1]
    grid = n // _BK
    full = lambda i: (0, 0)
    return pl.pallas_call(
        _gcn2_kernel,
        grid=(grid,),
        in_specs=[
            pl.BlockSpec((n, x.shape[1]), full),
            pl.BlockSpec((_BK, n), lambda i: (i, 0)),
            pl.BlockSpec(W1.shape, full),
            pl.BlockSpec((1, d_out), full),
            pl.BlockSpec(W2.shape, full),
            pl.BlockSpec((1, d_out), full),
        ],
        out_specs=pl.BlockSpec((n, d_out), full),
        scratch_shapes=[
            pltpu.VMEM((n, n), jnp.float32),
            pltpu.VMEM((n, 1), jnp.float32),
            pltpu.VMEM((n, d_out), jnp.float32),
        ],
        out_shape=jax.ShapeDtypeStruct((n, d_out), x.dtype),
    )(
        x,
        adjacency_matrix,
        W1,
        b1.reshape(1, -1),
        W2,
        b2.reshape(1, -1),
    )


# speedup vs baseline: 1.6628x; 1.6628x over previous
import jax
import jax.numpy as jnp
from jax.experimental import pallas as pl


def _gcn2_kernel(x_ref, a_ref, w1_ref, b1_ref, w2_ref, b2_ref, out_ref):
    a = a_ref[...]
    deg_row = jnp.sum(a, axis=0, keepdims=True) + 1.0
    s_row = jnp.where(deg_row > 0, jax.lax.rsqrt(deg_row), 0.0)
    s = jnp.transpose(s_row)

    def layer(h_in, w_ref, b_ref):
        h = jnp.dot(h_in, w_ref[...], preferred_element_type=jnp.float32)
        hs = s * h
        m = jax.lax.dot_general(
            a, hs, (((0,), (0,)), ((), ())),
            preferred_element_type=jnp.float32,
        ) + hs
        return jax.nn.relu(s * m + b_ref[...])

    h1 = layer(x_ref[...], w1_ref, b1_ref)
    out_ref[...] = layer(h1, w2_ref, b2_ref)


def kernel(x, adjacency_matrix, W1, b1, W2, b2):
    n, d_out = x.shape[0], W2.shape[1]
    return pl.pallas_call(
        _gcn2_kernel,
        out_shape=jax.ShapeDtypeStruct((n, d_out), x.dtype),
    )(
        x,
        adjacency_matrix,
        W1,
        b1.reshape(1, -1),
        W2,
        b2.reshape(1, -1),
    )
